# baseline (speedup 1.0000x reference)
import jax
import jax.numpy as jnp
from jax import lax
from jax.experimental import pallas as pl
from jax.experimental.pallas import tpu as pltpu
from jax.experimental.pallas import tpu_sc as plsc

_M = 1000000
_D = 16
_B = 16384

_NC = 2
_NS = 16
_NW = _NC * _NS

_W = 2048
_NFULL = 488
_SPECIAL = 488
_SPECIAL_W = 512
_SPECIAL_OWNER = _SPECIAL % _NW
_TAIL0 = 999936
_TAILN = _M - _TAIL0
_TAIL_OWNER = 9

_BPW = _B // _NW
_HITCAP = 2048
_YB = 64

_MESH = plsc.VectorSubcoreMesh(core_axis_name="c", subcore_axis_name="s")
_PARAMS = pltpu.CompilerParams(use_tc_tiling_on_sc=True, needs_layout_passes=False)
_IOTA = lambda: lax.iota(jnp.int32, 16)


def _pos_body(idx_hbm, pos_ref, idx_v, vals_v, sem):
    wid = lax.axis_index("s") * _NC + lax.axis_index("c")
    base = wid * _BPW
    pltpu.sync_copy(idx_hbm.at[pl.ds(wid * 4, 4)], idx_v)
    for j in range(4):
        for k in range(8):
            vals_v[j, pl.ds(k * 16, 16)] = base + j * 128 + k * 16 + _IOTA()
    copies = [
        pltpu.async_copy(vals_v.at[j], pos_ref.at[idx_v.at[j]], sem)
        for j in range(4)
    ]
    for c in copies:
        c.wait()


_sc_pos = pl.kernel(
    _pos_body,
    out_type=(),
    mesh=_MESH,
    compiler_params=pltpu.CompilerParams(use_tc_tiling_on_sc=False),
    scratch_types=[
        pltpu.VMEM((4, 128), jnp.int32),
        pltpu.VMEM((4, 128), jnp.int32),
        pltpu.SemaphoreType.DMA,
    ],
)


def _dense_body(xt_hbm, pos_hbm, y2d_hbm, yflat_hbm, xtail_hbm, out_hbm,
                tail_hbm, buf, buf2, pbuf, stage, tbuf,
                clist, slist, sublist, ystage, sbatch, nref,
                sem, sem2, psem, psem2, osem, osem2, ysem):
    wid = lax.axis_index("s") * _NC + lax.axis_index("c")

    def inits(i, _):
        slist[pl.ds(i * 16, 16)] = jnp.zeros((16,), jnp.int32)
        return 0

    lax.fori_loop(0, _HITCAP // 16, inits, 0)

    def work(b, poff, ncols):
        nref[0] = 0

        def collect(gi, _):
            base = poff + gi * 128
            m = pbuf[pl.ds(base, 16)] >= 0
            for q in range(1, 8):
                m = m | (pbuf[pl.ds(base + q * 16, 16)] >= 0)

            @pl.when(plsc.all_reduce_population_count(m)[0] > 0)
            def _():
                for q in range(8):
                    pv = pbuf[pl.ds(base + q * 16, 16)]
                    mq = pv >= 0
                    n = nref[0]
                    plsc.store_compressed(clist.at[pl.ds(n, 16)],
                                          gi * 128 + q * 16 + _IOTA(),
                                          mask=mq)
                    plsc.store_compressed(slist.at[pl.ds(n, 16)],
                                          lax.shift_right_logical(pv, 3),
                                          mask=mq)
                    plsc.store_compressed(sublist.at[pl.ds(n, 16)],
                                          pv & 7, mask=mq)
                    nref[0] = n + plsc.all_reduce_population_count(mq)[0]
            return 0

        lax.fori_loop(0, ncols // 128, collect, 0)
        n = nref[0]

        def load_sbatch(j0):
            for t in range(_YB // 16):
                sbatch[0, pl.ds(t * 16, 16)] = slist[pl.ds(j0 + t * 16, 16)]

        @pl.when(n > 0)
        def _():
            load_sbatch(0)
            pltpu.async_copy(y2d_hbm.at[sbatch.at[0]], ystage, ysem)

        def dblr(r, _):
            def dbl(c, _):
                b[r, pl.ds(c * 16, 16)] = b[r, pl.ds(c * 16, 16)] * 2.0
                return 0

            lax.fori_loop(0, ncols // 16, dbl, 0, unroll=8)
            return 0

        lax.fori_loop(0, _D, dblr, 0)

        def apply16(j0, nlim):
            cv = clist[pl.ds(j0, 16)]
            sv = sublist[pl.ds(j0, 16)]
            for k in range(16):
                @pl.when(j0 + k < nlim)
                def _(k=k, cv=cv, sv=sv):
                    sub = sv[k]
                    v = ystage[(j0 + k) % _YB, pl.ds(sub * 16, 16)]
                    plsc.store_scatter(
                        b, [_IOTA(), jnp.full((16,), cv[k], jnp.int32)],
                        v + v)

        @pl.when(n > 0)
        def _():
            pltpu.make_async_copy(y2d_hbm.at[sbatch.at[0]], ystage,
                                  ysem).wait()

        def batch(bi, _):
            def chunk(q, _):
                apply16(bi * _YB + q * 16, n)
                return 0

            lax.fori_loop(0, _YB // 16, chunk, 0)

            @pl.when((bi + 1) * _YB < n)
            def _():
                load_sbatch((bi + 1) * _YB)
                pltpu.sync_copy(y2d_hbm.at[sbatch.at[0]], ystage)
            return 0

        lax.fori_loop(0, (n + _YB - 1) // _YB, batch, 0)

    nslabs = (_NFULL - wid + _NW - 1) // _NW

    def slab_pair(s2, _):
        g0 = (2 * s2) * _NW + wid
        g1 = g0 + _NW
        c00 = pl.multiple_of(g0 * _W, 128)
        c01 = pl.multiple_of(g1 * _W, 128)
        have2 = 2 * s2 + 1 < nslabs

        gh0 = pltpu.async_copy(xt_hbm.at[:, pl.ds(c00, _W)], buf, sem)
        ph0 = pltpu.async_copy(pos_hbm.at[pl.ds(c00, _W)],
                               pbuf.at[pl.ds(0, _W)], psem)

        @pl.when(have2)
        def _():
            pltpu.async_copy(xt_hbm.at[:, pl.ds(c01, _W)], buf2, sem2)
            pltpu.async_copy(pos_hbm.at[pl.ds(c01, _W)],
                             pbuf.at[pl.ds(_W, _W)], psem2)

        gh0.wait()
        ph0.wait()
        work(buf, 0, _W)
        wh0 = pltpu.async_copy(buf, out_hbm.at[:, pl.ds(c00, _W)], osem)

        @pl.when(have2)
        def _():
            pltpu.make_async_copy(xt_hbm.at[:, pl.ds(c01, _W)], buf2,
                                  sem2).wait()
            pltpu.make_async_copy(pos_hbm.at[pl.ds(c01, _W)],
                                  pbuf.at[pl.ds(_W, _W)], psem2).wait()
            work(buf2, _W, _W)
            pltpu.async_copy(buf2, out_hbm.at[:, pl.ds(c01, _W)],
                             osem2).wait()

        wh0.wait()
        return 0

    lax.fori_loop(0, (nslabs + 1) // 2, slab_pair, 0)

    @pl.when(wid == _SPECIAL_OWNER)
    def _():
        c0 = pl.multiple_of(_SPECIAL * _W, 128)
        pltpu.sync_copy(xt_hbm.at[:, pl.ds(c0, _SPECIAL_W)],
                        buf.at[:, pl.ds(0, _SPECIAL_W)])
        pltpu.sync_copy(pos_hbm.at[pl.ds(c0, _SPECIAL_W)],
                        pbuf.at[pl.ds(0, _SPECIAL_W)])
        work(buf, 0, _SPECIAL_W)
        pltpu.sync_copy(buf.at[:, pl.ds(0, _SPECIAL_W)],
                        out_hbm.at[:, pl.ds(c0, _SPECIAL_W)])

    @pl.when(wid == _TAIL_OWNER)
    def _():
        pltpu.sync_copy(xtail_hbm, tbuf)
        ph = pltpu.async_copy(pos_hbm.at[pl.ds(_TAIL0, _TAILN)],
                              pbuf.at[pl.ds(0, _TAILN)], psem)

        def dblt(i, _):
            tbuf[pl.ds(i * 16, 16)] = tbuf[pl.ds(i * 16, 16)] * 2.0
            return 0

        lax.fori_loop(0, _TAILN * _D // 16, dblt, 0, unroll=8)
        ph.wait()

        def scant(ci, _):
            pv = pbuf[pl.ds(ci * 16, 16)]
            mq = pv >= 0

            @pl.when(plsc.all_reduce_population_count(mq)[0] > 0)
            def _():
                def cond(mq):
                    return plsc.all_reduce_population_count(mq)[0] > 0

                def body(mq):
                    lane = plsc.all_reduce_ffs(mq)[0]
                    onehot = _IOTA() == lane
                    pp = jnp.sum(jnp.where(onehot, pv, 0))
                    ev = pp * _D + _IOTA()
                    pltpu.async_copy(yflat_hbm.at[ev], stage, ysem).wait()
                    v = stage[...]
                    row = ci * 16 + lane
                    tbuf[pl.ds(row * _D, _D)] = v + v
                    return mq & jnp.logical_not(onehot)

                lax.while_loop(cond, body, mq)
            return 0

        lax.fori_loop(0, _TAILN // 16, scant, 0)
        pltpu.sync_copy(tbuf, tail_hbm)


_sc_dense = pl.kernel(
    _dense_body,
    out_type=(
        jax.ShapeDtypeStruct((_D, _M), jnp.float32),
        jax.ShapeDtypeStruct((_TAILN * _D,), jnp.float32),
    ),
    mesh=_MESH,
    compiler_params=_PARAMS,
    scratch_types=[
        pltpu.VMEM((_D, _W), jnp.float32),
        pltpu.VMEM((_D, _W), jnp.float32),
        pltpu.VMEM((2 * _W,), jnp.int32),
        pltpu.VMEM((_D,), jnp.float32),
        pltpu.VMEM((_TAILN * _D,), jnp.float32),
        pltpu.VMEM((_HITCAP,), jnp.int32),
        pltpu.VMEM((_HITCAP,), jnp.int32),
        pltpu.VMEM((_HITCAP,), jnp.int32),
        pltpu.VMEM((_YB, 128), jnp.float32),
        pltpu.VMEM((1, _YB), jnp.int32),
        pltpu.SMEM((1,), jnp.int32),
        pltpu.SemaphoreType.DMA,
        pltpu.SemaphoreType.DMA,
        pltpu.SemaphoreType.DMA,
        pltpu.SemaphoreType.DMA,
        pltpu.SemaphoreType.DMA,
        pltpu.SemaphoreType.DMA,
        pltpu.SemaphoreType.DMA,
    ],
)


def kernel(x, y, index):
    xt = x.T
    yflat = y.reshape(_B * _D)
    xtail = x[_TAIL0:].reshape(_TAILN * _D)
    pos0 = jnp.full((_M,), -1, jnp.int32)
    pos_ref = jax.new_ref(pos0)
    _sc_pos(index.reshape(_NW * 4, 128), pos_ref)
    pos = jax.freeze(pos_ref)
    y2d = yflat.reshape(_B * _D // 128, 128)
    out_t, tail = _sc_dense(xt, pos, y2d, yflat, xtail)
    out = out_t.T
    return lax.dynamic_update_slice(out, tail.reshape(_TAILN, _D), (_TAIL0, 0))


# speedup vs baseline: 3.7741x; 3.7741x over previous
import jax
import jax.numpy as jnp
from jax import lax
from jax.experimental import pallas as pl
from jax.experimental.pallas import tpu as pltpu
from jax.experimental.pallas import tpu_sc as plsc

_M = 1000000
_D = 16
_B = 16384

_NC = 2
_NS = 16
_NW = _NC * _NS

_W = 2048
_NFULL = 488
_SPECIAL = 488
_SPECIAL_W = 512
_SPECIAL_OWNER = _SPECIAL % _NW
_TAIL0 = 999936
_TAILN = _M - _TAIL0
_TAIL_OWNER = 9

_BPW = _B // _NW
_HITCAP = 2048
_YB = 32

_MESH = plsc.VectorSubcoreMesh(core_axis_name="c", subcore_axis_name="s")
_PARAMS = pltpu.CompilerParams(use_tc_tiling_on_sc=True, needs_layout_passes=False)
_IOTA = lambda: lax.iota(jnp.int32, 16)


def _pos_body(idx_hbm, pos_ref, idx_v, vals_v, sem):
    wid = lax.axis_index("s") * _NC + lax.axis_index("c")
    base = wid * _BPW
    pltpu.sync_copy(idx_hbm.at[pl.ds(wid * 4, 4)], idx_v)
    for j in range(4):
        for k in range(8):
            vals_v[j, pl.ds(k * 16, 16)] = base + j * 128 + k * 16 + _IOTA()
    copies = [
        pltpu.async_copy(vals_v.at[j], pos_ref.at[idx_v.at[j]], sem)
        for j in range(4)
    ]
    for c in copies:
        c.wait()


_sc_pos = pl.kernel(
    _pos_body,
    out_type=(),
    mesh=_MESH,
    compiler_params=pltpu.CompilerParams(use_tc_tiling_on_sc=False),
    scratch_types=[
        pltpu.VMEM((4, 128), jnp.int32),
        pltpu.VMEM((4, 128), jnp.int32),
        pltpu.SemaphoreType.DMA,
    ],
)


def _dense_body(xt_hbm, pos_hbm, yflat_hbm, xtail_hbm, out_hbm,
                tail_hbm, buf, buf2, pbuf, stage, tbuf,
                clist, plist, ystage, nref,
                sem, sem2, psem, psem2, osem, osem2, ysem):
    wid = lax.axis_index("s") * _NC + lax.axis_index("c")


    def work(b, poff, ncols):
        nref[0] = 0

        def collect(gi, _):
            base = poff + gi * 128
            m = pbuf[pl.ds(base, 16)] >= 0
            for q in range(1, 8):
                m = m | (pbuf[pl.ds(base + q * 16, 16)] >= 0)

            @pl.when(plsc.all_reduce_population_count(m)[0] > 0)
            def _():
                for q in range(8):
                    pv = pbuf[pl.ds(base + q * 16, 16)]
                    mq = pv >= 0
                    n = nref[0]
                    plsc.store_compressed(clist.at[pl.ds(n, 16)],
                                          gi * 128 + q * 16 + _IOTA(),
                                          mask=mq)
                    plsc.store_compressed(plist.at[pl.ds(n, 16)],
                                          pv, mask=mq)
                    nref[0] = n + plsc.all_reduce_population_count(mq)[0]
            return 0

        lax.fori_loop(0, ncols // 128, collect, 0)
        n = nref[0]

        def dblr(r, _):
            def dbl(c, _):
                b[r, pl.ds(c * 16, 16)] = b[r, pl.ds(c * 16, 16)] * 2.0
                return 0

            lax.fori_loop(0, ncols // 16, dbl, 0, unroll=8)
            return 0

        lax.fori_loop(0, _D, dblr, 0)

        def batch(bi, _):
            j0 = bi * _YB
            pv0 = plist[pl.ds(j0, 16)]
            pv1 = plist[pl.ds(j0 + 16, 16)]
            for k in range(_YB):
                pk = pv0[k] if k < 16 else pv1[k - 16]

                @pl.when(j0 + k < n)
                def _(pk=pk, k=k):
                    pltpu.async_copy(yflat_hbm.at[pl.ds(pk * _D, _D)],
                                     ystage.at[k], ysem)

            cnt = jnp.minimum(n - j0, _YB)

            def drain(i, _):
                pltpu.make_async_copy(yflat_hbm.at[pl.ds(0, _D)],
                                      ystage.at[0], ysem).wait()
                return 0

            lax.fori_loop(0, cnt, drain, 0)

            cv0 = clist[pl.ds(j0, 16)]
            cv1 = clist[pl.ds(j0 + 16, 16)]
            for k in range(_YB):
                ck = cv0[k] if k < 16 else cv1[k - 16]

                @pl.when(j0 + k < n)
                def _(ck=ck, k=k):
                    v = ystage[k, :]
                    plsc.store_scatter(
                        b, [_IOTA(), jnp.full((16,), ck, jnp.int32)],
                        v + v)
            return 0

        lax.fori_loop(0, (n + _YB - 1) // _YB, batch, 0)

    nslabs = (_NFULL - wid + _NW - 1) // _NW

    def slab_pair(s2, _):
        g0 = (2 * s2) * _NW + wid
        g1 = g0 + _NW
        c00 = pl.multiple_of(g0 * _W, 128)
        c01 = pl.multiple_of(g1 * _W, 128)
        have2 = 2 * s2 + 1 < nslabs

        gh0 = pltpu.async_copy(xt_hbm.at[:, pl.ds(c00, _W)], buf, sem)
        ph0 = pltpu.async_copy(pos_hbm.at[pl.ds(c00, _W)],
                               pbuf.at[pl.ds(0, _W)], psem)

        @pl.when(have2)
        def _():
            pltpu.async_copy(xt_hbm.at[:, pl.ds(c01, _W)], buf2, sem2)
            pltpu.async_copy(pos_hbm.at[pl.ds(c01, _W)],
                             pbuf.at[pl.ds(_W, _W)], psem2)

        gh0.wait()
        ph0.wait()
        work(buf, 0, _W)
        wh0 = pltpu.async_copy(buf, out_hbm.at[:, pl.ds(c00, _W)], osem)

        @pl.when(have2)
        def _():
            pltpu.make_async_copy(xt_hbm.at[:, pl.ds(c01, _W)], buf2,
                                  sem2).wait()
            pltpu.make_async_copy(pos_hbm.at[pl.ds(c01, _W)],
                                  pbuf.at[pl.ds(_W, _W)], psem2).wait()
            work(buf2, _W, _W)
            pltpu.async_copy(buf2, out_hbm.at[:, pl.ds(c01, _W)],
                             osem2).wait()

        wh0.wait()
        return 0

    lax.fori_loop(0, (nslabs + 1) // 2, slab_pair, 0)

    @pl.when(wid == _SPECIAL_OWNER)
    def _():
        c0 = pl.multiple_of(_SPECIAL * _W, 128)
        pltpu.sync_copy(xt_hbm.at[:, pl.ds(c0, _SPECIAL_W)],
                        buf.at[:, pl.ds(0, _SPECIAL_W)])
        pltpu.sync_copy(pos_hbm.at[pl.ds(c0, _SPECIAL_W)],
                        pbuf.at[pl.ds(0, _SPECIAL_W)])
        work(buf, 0, _SPECIAL_W)
        pltpu.sync_copy(buf.at[:, pl.ds(0, _SPECIAL_W)],
                        out_hbm.at[:, pl.ds(c0, _SPECIAL_W)])

    @pl.when(wid == _TAIL_OWNER)
    def _():
        pltpu.sync_copy(xtail_hbm, tbuf)
        ph = pltpu.async_copy(pos_hbm.at[pl.ds(_TAIL0, _TAILN)],
                              pbuf.at[pl.ds(0, _TAILN)], psem)

        def dblt(i, _):
            tbuf[pl.ds(i * 16, 16)] = tbuf[pl.ds(i * 16, 16)] * 2.0
            return 0

        lax.fori_loop(0, _TAILN * _D // 16, dblt, 0, unroll=8)
        ph.wait()

        def scant(ci, _):
            pv = pbuf[pl.ds(ci * 16, 16)]
            mq = pv >= 0

            @pl.when(plsc.all_reduce_population_count(mq)[0] > 0)
            def _():
                def cond(mq):
                    return plsc.all_reduce_population_count(mq)[0] > 0

                def body(mq):
                    lane = plsc.all_reduce_ffs(mq)[0]
                    onehot = _IOTA() == lane
                    pp = jnp.sum(jnp.where(onehot, pv, 0))
                    ev = pp * _D + _IOTA()
                    pltpu.async_copy(yflat_hbm.at[ev], stage, ysem).wait()
                    v = stage[...]
                    row = ci * 16 + lane
                    tbuf[pl.ds(row * _D, _D)] = v + v
                    return mq & jnp.logical_not(onehot)

                lax.while_loop(cond, body, mq)
            return 0

        lax.fori_loop(0, _TAILN // 16, scant, 0)
        pltpu.sync_copy(tbuf, tail_hbm)


_sc_dense = pl.kernel(
    _dense_body,
    out_type=(
        jax.ShapeDtypeStruct((_D, _M), jnp.float32),
        jax.ShapeDtypeStruct((_TAILN * _D,), jnp.float32),
    ),
    mesh=_MESH,
    compiler_params=_PARAMS,
    scratch_types=[
        pltpu.VMEM((_D, _W), jnp.float32),
        pltpu.VMEM((_D, _W), jnp.float32),
        pltpu.VMEM((2 * _W,), jnp.int32),
        pltpu.VMEM((_D,), jnp.float32),
        pltpu.VMEM((_TAILN * _D,), jnp.float32),
        pltpu.VMEM((_HITCAP,), jnp.int32),
        pltpu.VMEM((_HITCAP,), jnp.int32),
        pltpu.VMEM((_YB, _D), jnp.float32),
        pltpu.SMEM((1,), jnp.int32),
        pltpu.SemaphoreType.DMA,
        pltpu.SemaphoreType.DMA,
        pltpu.SemaphoreType.DMA,
        pltpu.SemaphoreType.DMA,
        pltpu.SemaphoreType.DMA,
        pltpu.SemaphoreType.DMA,
        pltpu.SemaphoreType.DMA,
    ],
)


def kernel(x, y, index):
    xt = x.T
    yflat = y.reshape(_B * _D)
    xtail = x[_TAIL0:].reshape(_TAILN * _D)
    pos0 = jnp.full((_M,), -1, jnp.int32)
    pos_ref = jax.new_ref(pos0)
    _sc_pos(index.reshape(_NW * 4, 128), pos_ref)
    pos = jax.freeze(pos_ref)
    out_t, tail = _sc_dense(xt, pos, yflat, xtail)
    out = out_t.T
    return lax.dynamic_update_slice(out, tail.reshape(_TAILN, _D), (_TAIL0, 0))
